# Initial kernel scaffold; baseline (speedup 1.0000x reference)
import jax
import jax.numpy as jnp
from jax.experimental import pallas as pl


def kernel(batch_X, batch_y, batch_N, in_embedding, out_embedding):
    raise NotImplementedError("write your pallas kernel here")


# speedup vs baseline: 4.0346x; 4.0346x over previous
import jax
import jax.numpy as jnp
from jax import lax
from jax.experimental import pallas as pl
from jax.experimental.pallas import tpu as pltpu
from jax.experimental.pallas import tpu_sc as plsc

_DIM = 128
_B = 4096
_W = 20
_K = 50
_NC = 2
_NS = 16
_L = 16
_NW = _NC * _NS
_BPW = _B // _NW
_CB = 2
_NCHUNK = _BPW // _CB
_YC = _CB * _W
_KC = _CB * _K
_NSEG = _DIM // _L
_NEG_SCALE = 1.0 / (_B * _K)


def _log_sigmoid(v):
    u = jnp.exp(-jnp.abs(v))
    s = u / (u + 2.0)
    s2 = s * s
    poly = 1.0 + s2 * (
        (1.0 / 3.0)
        + s2 * ((1.0 / 5.0) + s2 * ((1.0 / 7.0) + s2 * ((1.0 / 9.0) + s2 * (1.0 / 11.0))))
    )
    return jnp.minimum(v, 0.0) - 2.0 * s * poly


def _iota():
    return jnp.arange(_L, dtype=jnp.int32)


def _pack_dots(buf, xv, rows_per_elem, g, nval):
    d = jnp.zeros((_L,), jnp.float32)
    idx = _iota()
    for r16 in range(nval):
        row = g * _L + r16
        e = row // rows_per_elem
        p = buf[row, pl.ds(0, _L)] * xv[e][0]
        for j in range(1, _NSEG):
            p = p + buf[row, pl.ds(_L * j, _L)] * xv[e][j]
        d = jnp.where(idx == r16, jnp.full((_L,), jnp.sum(p)), d)
    return d


def _sgns_body(bx_hbm, by_hbm, bn_hbm, in_emb, out_emb, out_hbm,
               bxv, byv, bnv, xall, ybuf, nbuf, accp, semx, semy, semn):
    wid = lax.axis_index("s") * _NC + lax.axis_index("c")
    base = wid * _BPW

    pltpu.sync_copy(bx_hbm.at[pl.ds(base, _BPW)], bxv)
    pltpu.sync_copy(by_hbm.at[pl.ds(base * _W, _BPW * _W)], byv.at[pl.ds(0, _BPW * _W)])
    pltpu.sync_copy(bn_hbm.at[pl.ds(wid * _NCHUNK, _NCHUNK)], bnv)
    pltpu.async_copy(in_emb.at[bxv], xall, semx).wait()

    zero16 = jnp.zeros((_L,), jnp.float32)
    idx = _iota()

    def chunk(c, carry):
        ay, an = carry
        cy = pltpu.async_copy(out_emb.at[byv.at[pl.ds(c * _YC, _YC)]], ybuf, semy)
        cn = pltpu.async_copy(out_emb.at[bnv.at[c]], nbuf, semn)
        cy.wait()
        cn.wait()

        xv = [[xall[c * _CB + e, pl.ds(_L * j, _L)] for j in range(_NSEG)]
              for e in range(_CB)]

        for g in range((_YC + _L - 1) // _L):
            nval = min(_L, _YC - g * _L)
            d = _pack_dots(ybuf, xv, _W, g, nval)
            mvec = byv[pl.ds(c * _YC + g * _L, _L)]
            ok = mvec != 0
            if nval < _L:
                ok = ok & (idx < nval)
            ay = ay + jnp.where(ok, _log_sigmoid(d), zero16)

        for g in range((_KC + _L - 1) // _L):
            nval = min(_L, _KC - g * _L)
            d = _pack_dots(nbuf, xv, _K, g, nval)
            val = _log_sigmoid(-d)
            if nval < _L:
                val = jnp.where(idx < nval, val, zero16)
            an = an + val
        return ay, an

    acc_y, acc_n = lax.fori_loop(0, _NCHUNK, chunk, (zero16, zero16))

    accp[...] = acc_y + acc_n * jnp.float32(_NEG_SCALE)
    pltpu.sync_copy(accp, out_hbm.at[wid])


@jax.jit
def _sgns_partials(batch_X, by_flat, bn2, in_embedding, out_embedding):
    mesh = plsc.VectorSubcoreMesh(core_axis_name="c", subcore_axis_name="s")
    return pl.kernel(
        _sgns_body,
        out_type=jax.ShapeDtypeStruct((_NW, _L), jnp.float32),
        mesh=mesh,
        compiler_params=pltpu.CompilerParams(needs_layout_passes=False),
        scratch_types=[
            pltpu.VMEM((_BPW,), jnp.int32),
            pltpu.VMEM((_BPW * _W + _L,), jnp.int32),
            pltpu.VMEM((_NCHUNK, _KC), jnp.int32),
            pltpu.VMEM((_BPW, _DIM), jnp.float32),
            pltpu.VMEM((_YC, _DIM), jnp.float32),
            pltpu.VMEM((_KC, _DIM), jnp.float32),
            pltpu.VMEM((_L,), jnp.float32),
            pltpu.SemaphoreType.DMA,
            pltpu.SemaphoreType.DMA,
            pltpu.SemaphoreType.DMA,
        ],
    )(batch_X, by_flat, bn2, in_embedding, out_embedding)


def kernel(batch_X, batch_y, batch_N, in_embedding, out_embedding):
    by_flat = batch_y.reshape(_B * _W)
    bn2 = batch_N.reshape(_B * _K // _KC, _KC)
    parts = _sgns_partials(batch_X, by_flat, bn2, in_embedding, out_embedding)
    return -jnp.sum(parts)


# speedup vs baseline: 4.3185x; 1.0704x over previous
import jax
import jax.numpy as jnp
from jax import lax
from jax.experimental import pallas as pl
from jax.experimental.pallas import tpu as pltpu
from jax.experimental.pallas import tpu_sc as plsc

_DIM = 128
_B = 4096
_W = 20
_K = 50
_NC = 2
_NS = 16
_L = 16
_NW = _NC * _NS
_BPW = _B // _NW
_CB = 2
_NCHUNK = _BPW // _CB
_YC = _CB * _W
_KC = _CB * _K
_NSEG = _DIM // _L
_NEG_SCALE = 1.0 / (_B * _K)


def _log_sigmoid(v):
    u = jnp.exp(-jnp.abs(v))
    s = u / (u + 2.0)
    s2 = s * s
    poly = 1.0 + s2 * (
        (1.0 / 3.0)
        + s2 * ((1.0 / 5.0) + s2 * ((1.0 / 7.0) + s2 * ((1.0 / 9.0) + s2 * (1.0 / 11.0))))
    )
    return jnp.minimum(v, 0.0) - 2.0 * s * poly


def _iota():
    return jnp.arange(_L, dtype=jnp.int32)


def _tree_sum(terms):
    ts = list(terms)
    while len(ts) > 1:
        ts = [ts[i] + ts[i + 1] for i in range(0, len(ts) - 1, 2)] + (
            [ts[-1]] if len(ts) % 2 else [])
    return ts[0]


def _row_partials(buf, dots, xall, c, rows_per_elem):
    for e in range(_CB):
        xv = [xall[c * _CB + e, pl.ds(_L * j, _L)] for j in range(_NSEG)]
        for r in range(rows_per_elem):
            row = e * rows_per_elem + r
            p = _tree_sum([buf[row, pl.ds(_L * j, _L)] * xv[j]
                           for j in range(_NSEG)])
            dots[pl.ds(row * _L, _L)] = p


def _transpose_sum(dots, g):
    off = _iota() * _L + g * (_L * _L)
    return _tree_sum([plsc.load_gather(dots, [off + l]) for l in range(_L)])


def _sgns_body(bx_hbm, by_hbm, bn_hbm, in_emb, out_emb, out_hbm,
               bxv, byv, bnv, xall, ybuf, nbuf, dots_y, dots_n, accp,
               semx, semy, semn):
    wid = lax.axis_index("s") * _NC + lax.axis_index("c")
    base = wid * _BPW

    pltpu.sync_copy(bx_hbm.at[pl.ds(base, _BPW)], bxv)
    pltpu.sync_copy(by_hbm.at[pl.ds(base * _W, _BPW * _W)], byv.at[pl.ds(0, _BPW * _W)])
    pltpu.sync_copy(bn_hbm.at[pl.ds(wid * _NCHUNK, _NCHUNK)], bnv)
    pltpu.async_copy(in_emb.at[bxv], xall, semx).wait()

    zero16 = jnp.zeros((_L,), jnp.float32)
    idx = _iota()

    def chunk(c, carry):
        ay, an = carry
        cy = pltpu.async_copy(out_emb.at[byv.at[pl.ds(c * _YC, _YC)]], ybuf, semy)
        cn = pltpu.async_copy(out_emb.at[bnv.at[c]], nbuf, semn)
        cy.wait()
        cn.wait()

        _row_partials(ybuf, dots_y, xall, c, _W)
        _row_partials(nbuf, dots_n, xall, c, _K)

        for g in range((_YC + _L - 1) // _L):
            nval = min(_L, _YC - g * _L)
            d = _transpose_sum(dots_y, g)
            mvec = byv[pl.ds(c * _YC + g * _L, _L)]
            ok = mvec != 0
            if nval < _L:
                ok = ok & (idx < nval)
            ay = ay + jnp.where(ok, _log_sigmoid(d), zero16)

        for g in range((_KC + _L - 1) // _L):
            nval = min(_L, _KC - g * _L)
            d = _transpose_sum(dots_n, g)
            val = _log_sigmoid(-d)
            if nval < _L:
                val = jnp.where(idx < nval, val, zero16)
            an = an + val
        return ay, an

    acc_y, acc_n = lax.fori_loop(0, _NCHUNK, chunk, (zero16, zero16))

    accp[...] = acc_y + acc_n * jnp.float32(_NEG_SCALE)
    pltpu.sync_copy(accp, out_hbm.at[wid])


@jax.jit
def _sgns_partials(batch_X, by_flat, bn2, in_embedding, out_embedding):
    mesh = plsc.VectorSubcoreMesh(core_axis_name="c", subcore_axis_name="s")
    return pl.kernel(
        _sgns_body,
        out_type=jax.ShapeDtypeStruct((_NW, _L), jnp.float32),
        mesh=mesh,
        compiler_params=pltpu.CompilerParams(needs_layout_passes=False),
        scratch_types=[
            pltpu.VMEM((_BPW,), jnp.int32),
            pltpu.VMEM((_BPW * _W + _L,), jnp.int32),
            pltpu.VMEM((_NCHUNK, _KC), jnp.int32),
            pltpu.VMEM((_BPW, _DIM), jnp.float32),
            pltpu.VMEM((_YC, _DIM), jnp.float32),
            pltpu.VMEM((_KC, _DIM), jnp.float32),
            pltpu.VMEM((((_YC + _L - 1) // _L) * _L * _L,), jnp.float32),
            pltpu.VMEM((((_KC + _L - 1) // _L) * _L * _L,), jnp.float32),
            pltpu.VMEM((_L,), jnp.float32),
            pltpu.SemaphoreType.DMA,
            pltpu.SemaphoreType.DMA,
            pltpu.SemaphoreType.DMA,
        ],
    )(batch_X, by_flat, bn2, in_embedding, out_embedding)


def kernel(batch_X, batch_y, batch_N, in_embedding, out_embedding):
    by_flat = batch_y.reshape(_B * _W)
    bn2 = batch_N.reshape(_B * _K // _KC, _KC)
    parts = _sgns_partials(batch_X, by_flat, bn2, in_embedding, out_embedding)
    return -jnp.sum(parts)


# speedup vs baseline: 5.2693x; 1.2202x over previous
import jax
import jax.numpy as jnp
from jax import lax
from jax.experimental import pallas as pl
from jax.experimental.pallas import tpu as pltpu
from jax.experimental.pallas import tpu_sc as plsc

_DIM = 128
_B = 4096
_W = 20
_K = 50
_NC = 2
_NS = 16
_L = 16
_NW = _NC * _NS
_BPW = _B // _NW
_CB = 2
_NCHUNK = _BPW // _CB
_YC = _CB * _W
_KC = _CB * _K
_NSEG = _DIM // _L
_NEG_SCALE = 1.0 / (_B * _K)


def _log_sigmoid(v):
    u = jnp.exp(-jnp.abs(v))
    s = u / (u + 2.0)
    s2 = s * s
    poly = 1.0 + s2 * (
        (1.0 / 3.0)
        + s2 * ((1.0 / 5.0) + s2 * ((1.0 / 7.0) + s2 * ((1.0 / 9.0) + s2 * (1.0 / 11.0))))
    )
    return jnp.minimum(v, 0.0) - 2.0 * s * poly


def _iota():
    return jnp.arange(_L, dtype=jnp.int32)


def _tree_sum(terms):
    ts = list(terms)
    while len(ts) > 1:
        ts = [ts[i] + ts[i + 1] for i in range(0, len(ts) - 1, 2)] + (
            [ts[-1]] if len(ts) % 2 else [])
    return ts[0]


def _row_partials(buf, dots, xall, c, rows_per_elem):
    for e in range(_CB):
        xv = [xall[c * _CB + e, pl.ds(_L * j, _L)] for j in range(_NSEG)]
        for r in range(rows_per_elem):
            row = e * rows_per_elem + r
            p = _tree_sum([buf[row, pl.ds(_L * j, _L)] * xv[j]
                           for j in range(_NSEG)])
            dots[pl.ds(row * _L, _L)] = p


def _transpose_sum(dots, g):
    off = _iota() * _L + g * (_L * _L)
    return _tree_sum([plsc.load_gather(dots, [off + l]) for l in range(_L)])


def _sgns_body(bx_hbm, by_hbm, bn_hbm, in_emb, out_emb, out_hbm,
               bxv, byv, bnv, xall, ybuf, nbuf, dots_y, dots_n, accp,
               semx, semy0, semn0, semy1, semn1):
    wid = lax.axis_index("s") * _NC + lax.axis_index("c")
    base = wid * _BPW

    pltpu.sync_copy(bx_hbm.at[pl.ds(base, _BPW)], bxv)
    pltpu.sync_copy(by_hbm.at[pl.ds(base * _W, _BPW * _W)], byv.at[pl.ds(0, _BPW * _W)])
    pltpu.sync_copy(bn_hbm.at[pl.ds(wid * _NCHUNK, _NCHUNK)], bnv)
    pltpu.async_copy(in_emb.at[bxv], xall, semx).wait()

    zero16 = jnp.zeros((_L,), jnp.float32)
    idx = _iota()
    sems = ((semy0, semn0), (semy1, semn1))

    def issue(c, buf):
        sy, sn = sems[buf]
        pltpu.async_copy(out_emb.at[byv.at[pl.ds(c * _YC, _YC)]],
                         ybuf.at[buf], sy)
        pltpu.async_copy(out_emb.at[bnv.at[c]], nbuf.at[buf], sn)

    def drain(c, buf):
        sy, sn = sems[buf]
        pltpu.make_async_copy(out_emb.at[byv.at[pl.ds(c * _YC, _YC)]],
                              ybuf.at[buf], sy).wait()
        pltpu.make_async_copy(out_emb.at[bnv.at[c]], nbuf.at[buf], sn).wait()

    def compute(c, buf, ay, an):
        _row_partials(ybuf.at[buf], dots_y, xall, c, _W)
        _row_partials(nbuf.at[buf], dots_n, xall, c, _K)

        for g in range((_YC + _L - 1) // _L):
            nval = min(_L, _YC - g * _L)
            d = _transpose_sum(dots_y, g)
            mvec = byv[pl.ds(c * _YC + g * _L, _L)]
            ok = mvec != 0
            if nval < _L:
                ok = ok & (idx < nval)
            ay = ay + jnp.where(ok, _log_sigmoid(d), zero16)

        for g in range((_KC + _L - 1) // _L):
            nval = min(_L, _KC - g * _L)
            d = _transpose_sum(dots_n, g)
            val = _log_sigmoid(-d)
            if nval < _L:
                val = jnp.where(idx < nval, val, zero16)
            an = an + val
        return ay, an

    issue(0, 0)

    def pair(i, carry):
        ay, an = carry
        issue(2 * i + 1, 1)
        drain(2 * i, 0)
        ay, an = compute(2 * i, 0, ay, an)

        @pl.when(i < _NCHUNK // 2 - 1)
        def _():
            issue(2 * i + 2, 0)

        drain(2 * i + 1, 1)
        ay, an = compute(2 * i + 1, 1, ay, an)
        return ay, an

    acc_y, acc_n = lax.fori_loop(0, _NCHUNK // 2, pair, (zero16, zero16))

    accp[...] = acc_y + acc_n * jnp.float32(_NEG_SCALE)
    pltpu.sync_copy(accp, out_hbm.at[wid])


@jax.jit
def _sgns_partials(batch_X, by_flat, bn2, in_embedding, out_embedding):
    mesh = plsc.VectorSubcoreMesh(core_axis_name="c", subcore_axis_name="s")
    return pl.kernel(
        _sgns_body,
        out_type=jax.ShapeDtypeStruct((_NW, _L), jnp.float32),
        mesh=mesh,
        compiler_params=pltpu.CompilerParams(needs_layout_passes=False),
        scratch_types=[
            pltpu.VMEM((_BPW,), jnp.int32),
            pltpu.VMEM((_BPW * _W + _L,), jnp.int32),
            pltpu.VMEM((_NCHUNK, _KC), jnp.int32),
            pltpu.VMEM((_BPW, _DIM), jnp.float32),
            pltpu.VMEM((2, _YC, _DIM), jnp.float32),
            pltpu.VMEM((2, _KC, _DIM), jnp.float32),
            pltpu.VMEM((((_YC + _L - 1) // _L) * _L * _L,), jnp.float32),
            pltpu.VMEM((((_KC + _L - 1) // _L) * _L * _L,), jnp.float32),
            pltpu.VMEM((_L,), jnp.float32),
            pltpu.SemaphoreType.DMA,
            pltpu.SemaphoreType.DMA,
            pltpu.SemaphoreType.DMA,
            pltpu.SemaphoreType.DMA,
            pltpu.SemaphoreType.DMA,
        ],
    )(batch_X, by_flat, bn2, in_embedding, out_embedding)


def kernel(batch_X, batch_y, batch_N, in_embedding, out_embedding):
    by_flat = batch_y.reshape(_B * _W)
    bn2 = batch_N.reshape(_B * _K // _KC, _KC)
    parts = _sgns_partials(batch_X, by_flat, bn2, in_embedding, out_embedding)
    return -jnp.sum(parts)
